# baseline (speedup 1.0000x reference)
import jax
import jax.numpy as jnp
import numpy as np
from jax.experimental import pallas as pl

_T = 512
_DH = 64
_R = 8
_U = _T // _R
_HI = jax.lax.Precision.HIGHEST


def _res(x2, r):
    return x2[:, r * _DH:(r + 1) * _DH]


def _shift_up(x):
    return jnp.concatenate([x[1:], jnp.zeros((1, _DH), jnp.float32)], axis=0)


def _shift_dn(x):
    return jnp.concatenate([jnp.zeros((1, _DH), jnp.float32), x[:-1]], axis=0)


def _one_head(q2, k2, v2):
    scale = np.float32(1.0 / np.sqrt(_DH))

    vs2 = jnp.sum(v2, axis=0, keepdims=True)
    vsum = _res(vs2, 0)
    for r in range(1, _R):
        vsum = vsum + _res(vs2, r)

    outs = []
    for r in range(_R):
        qr = _res(q2, r)
        kr = _res(k2, r)
        vr = _res(v2, r)
        s = jax.lax.dot_general(
            qr, kr, dimension_numbers=(((1,), (1,)), ((), ())),
            precision=_HI, preferred_element_type=jnp.float32)
        e = jnp.exp(scale * s) - 1.0
        den = jnp.sum(e, axis=1, keepdims=True)
        num = jax.lax.dot_general(
            e, vr, dimension_numbers=(((1,), (0,)), ((), ())),
            precision=_HI, preferred_element_type=jnp.float32)
        for d in (1, 2):
            rp = r + d
            if rp < _R:
                kp, vp = _res(k2, rp), _res(v2, rp)
            else:
                kp, vp = _shift_up(_res(k2, rp - _R)), _shift_up(_res(v2, rp - _R))
            wf = jnp.exp(scale * jnp.sum(qr * kp, axis=1, keepdims=True)) - 1.0
            num += wf * vp
            den += wf
            rm = r - d
            if rm >= 0:
                km, vm = _res(k2, rm), _res(v2, rm)
            else:
                km, vm = _shift_dn(_res(k2, rm + _R)), _shift_dn(_res(v2, rm + _R))
            wb = jnp.exp(scale * jnp.sum(qr * km, axis=1, keepdims=True)) - 1.0
            num += wb * vm
            den += wb
        outs.append((num + vsum) / (den + np.float32(_T)))

    return jnp.stack(outs, axis=1)


def _pair_kernel(q_ref, k_ref, v_ref, o_ref):
    o0 = _one_head(q_ref[0, 0, 0], k_ref[0, 0, 0], v_ref[0, 0, 0])
    o1 = _one_head(q_ref[0, 1, 0], k_ref[0, 1, 0], v_ref[0, 0, 1])
    o_ref[0, :, :, :] = jnp.concatenate([o0, o1], axis=2)


@jax.jit
def _run(q, k, v):
    B, N, H, T, Dh = q.shape
    qv = q.reshape(B, N, H, _U, _R * Dh)
    kv = k.reshape(B, N, H, _U, _R * Dh)
    vv = v.reshape(B, N, H, _U, _R * Dh)
    J = H // 2
    qk_spec = pl.BlockSpec((1, 2, 1, _U, _R * Dh),
                           lambda b, n, j: (b, j, n, 0, 0))
    v_spec = pl.BlockSpec((1, 1, 2, _U, _R * Dh),
                          lambda b, n, j: (b, n, j, 0, 0))
    out_spec = pl.BlockSpec((1, _U, _R, 2 * Dh),
                            lambda b, n, j: (b, 0, 0, n * J + j))
    out = pl.pallas_call(
        _pair_kernel,
        grid=(B, N, J),
        in_specs=[qk_spec, qk_spec, v_spec],
        out_specs=out_spec,
        out_shape=jax.ShapeDtypeStruct((B, _U, _R, N * H * Dh), jnp.float32),
    )(qv, kv, vv)
    return out.reshape(B, T, N, H * Dh)


def kernel(q, k, v, dims):
    return _run(q, k, v)


# speedup vs baseline: 3.2165x; 3.2165x over previous
import jax
import jax.numpy as jnp
import numpy as np
from jax.experimental import pallas as pl

_T = 512
_DH = 64
_STRIDE = 7
_LOCAL = 4


def _mask_scaled():
    t = np.arange(_T)
    dlt = np.abs(t[:, None] - t[None, :])
    m = (dlt <= _LOCAL // 2) | (dlt % (_STRIDE + 1) == 0)
    return (m.astype(np.float32) / np.sqrt(_DH)).astype(np.float32)


def _one_head(q, k, v, msk):
    s = jax.lax.dot_general(
        q, k, dimension_numbers=(((1,), (1,)), ((), ())),
        preferred_element_type=jnp.float32)
    ee = jnp.exp(s * msk) - 1.0
    va = jnp.concatenate([v, jnp.ones((_T, 1), jnp.float32)], axis=1)
    na = jax.lax.dot_general(
        ee, va, dimension_numbers=(((1,), (0,)), ((), ())),
        preferred_element_type=jnp.float32)
    vsum = jnp.sum(v, axis=0, keepdims=True)
    return (na[:, :_DH] + vsum) / (na[:, _DH:] + np.float32(_T))


def _pair_kernel(m_ref, q_ref, k_ref, v_ref, o_ref):
    msk = m_ref[...]
    o0 = _one_head(q_ref[0, 0, 0], k_ref[0, 0, 0], v_ref[0, 0, 0], msk)
    o1 = _one_head(q_ref[0, 1, 0], k_ref[0, 1, 0], v_ref[0, 0, 1], msk)
    o_ref[0, :, :] = jnp.concatenate([o0, o1], axis=1)


@jax.jit
def _run(q, k, v):
    B, N, H, T, Dh = q.shape
    J = H // 2
    msk = jnp.asarray(_mask_scaled())
    m_spec = pl.BlockSpec((T, T), lambda b, n, j: (0, 0))
    qk_spec = pl.BlockSpec((1, 2, 1, T, Dh), lambda b, n, j: (b, j, n, 0, 0))
    v_spec = pl.BlockSpec((1, 1, 2, T, Dh), lambda b, n, j: (b, n, j, 0, 0))
    out_spec = pl.BlockSpec((1, T, 2 * Dh), lambda b, n, j: (b, 0, n * J + j))
    out = pl.pallas_call(
        _pair_kernel,
        grid=(B, N, J),
        in_specs=[m_spec, qk_spec, qk_spec, v_spec],
        out_specs=out_spec,
        out_shape=jax.ShapeDtypeStruct((B, T, N * H * Dh), jnp.float32),
    )(msk, q, k, v)
    return out.reshape(B, T, N, H * Dh)


def kernel(q, k, v, dims):
    return _run(q, k, v)


# speedup vs baseline: 5.7150x; 1.7768x over previous
import jax
import jax.numpy as jnp
import numpy as np
from jax.experimental import pallas as pl

_T = 512
_DH = 64
_STRIDE = 7
_LOCAL = 4


def _mask_scaled():
    t = np.arange(_T)
    dlt = np.abs(t[:, None] - t[None, :])
    m = (dlt <= _LOCAL // 2) | (dlt % (_STRIDE + 1) == 0)
    return (m.astype(np.float32) / np.sqrt(_DH)).astype(np.float32)


def _one_head_t(qt, kt, vt, msk):
    s = jax.lax.dot_general(
        qt, kt, dimension_numbers=(((0,), (0,)), ((), ())),
        preferred_element_type=jnp.float32)
    ee = jnp.exp(s * msk) - 1.0
    va = jnp.concatenate([vt, jnp.ones((1, _T), jnp.float32)], axis=0)
    na = jax.lax.dot_general(
        va, ee, dimension_numbers=(((1,), (1,)), ((), ())),
        preferred_element_type=jnp.float32)
    vsum = jnp.sum(vt, axis=1, keepdims=True)
    return (na[:_DH] + vsum) / (na[_DH:] + np.float32(_T))


def _pair_kernel(m_ref, q_ref, k_ref, v_ref, o_ref):
    msk = m_ref[...]
    o0 = _one_head_t(q_ref[0, 0, 0], k_ref[0, 0, 0], v_ref[0, 0, 0], msk)
    o1 = _one_head_t(q_ref[0, 1, 0], k_ref[0, 1, 0], v_ref[0, 0, 1], msk)
    o_ref[0, :, :] = jnp.concatenate([o0, o1], axis=0)


@jax.jit
def _run(q, k, v):
    B, N, H, T, Dh = q.shape
    J = H // 2
    qt = jnp.swapaxes(q, 3, 4)
    kt = jnp.swapaxes(k, 3, 4)
    vt = jnp.swapaxes(v, 3, 4)
    msk = jnp.asarray(_mask_scaled())
    m_spec = pl.BlockSpec((T, T), lambda b, n, j: (0, 0))
    qk_spec = pl.BlockSpec((1, 2, 1, Dh, T), lambda b, n, j: (b, j, n, 0, 0))
    v_spec = pl.BlockSpec((1, 1, 2, Dh, T), lambda b, n, j: (b, n, j, 0, 0))
    out_spec = pl.BlockSpec((1, 2 * Dh, T), lambda b, n, j: (b, n * J + j, 0))
    out = pl.pallas_call(
        _pair_kernel,
        grid=(B, N, J),
        in_specs=[m_spec, qk_spec, qk_spec, v_spec],
        out_specs=out_spec,
        out_shape=jax.ShapeDtypeStruct((B, N * H * Dh, T), jnp.float32),
    )(msk, qt, kt, vt)
    return jnp.swapaxes(out, 1, 2).reshape(B, T, N, H * Dh)


def kernel(q, k, v, dims):
    return _run(q, k, v)
